# Initial kernel scaffold; baseline (speedup 1.0000x reference)
import jax
import jax.numpy as jnp
from jax.experimental import pallas as pl


def kernel(hidden, token_idxs, pooled_entities, W_align, b_align, W_out, b_out):
    raise NotImplementedError("write your pallas kernel here")


# speedup vs baseline: 1.3082x; 1.3082x over previous
import functools

import jax
import jax.numpy as jnp
from jax.experimental import pallas as pl
from jax.experimental.pallas import tpu as pltpu

_BS = 512


def _pool_body(nsb, t_spans, spans_ref, hid_ref, w1_ref, w2_ref, pe_ref, ba_ref,
               scores_ref, m_ref, d_ref, pooled_ref, acc_ref, m_s, d_s):
    b = pl.program_id(0)
    sb = pl.program_id(1)

    @pl.when(sb == 0)
    def _init():
        acc_ref[...] = jnp.zeros_like(acc_ref)
        m_s[0] = -jnp.inf
        d_s[0] = 0.0

    hid = hid_ref[0]
    sc = jnp.dot(hid, w2_ref[...], preferred_element_type=jnp.float32)
    pedot = jnp.dot(pe_ref[0], w1_ref[...], preferred_element_type=jnp.float32)
    sc = sc + pedot + ba_ref[0, 0]

    pos = sb * _BS + jax.lax.broadcasted_iota(jnp.int32, (_BS, 1), 0)
    mask = jnp.zeros((_BS, 1), dtype=jnp.bool_)
    for t in range(t_spans):
        s0 = spans_ref[b * 2 * t_spans + 2 * t]
        e0 = spans_ref[b * 2 * t_spans + 2 * t + 1]
        mask = mask | ((pos >= s0) & (pos < e0))

    scores_ref[...] = sc[None]

    sc_masked = jnp.where(mask, sc, -jnp.inf)
    m_old = m_s[0]
    m_new = jnp.maximum(m_old, jnp.max(sc_masked))
    m_safe = jnp.where(jnp.isfinite(m_new), m_new, 0.0)
    scale = jnp.exp(m_old - m_safe)
    e = jnp.where(mask, jnp.exp(sc - m_safe), 0.0)
    d_new = d_s[0] * scale + jnp.sum(e)
    contrib = jax.lax.dot_general(e, hid, (((0,), (0,)), ((), ())),
                                  preferred_element_type=jnp.float32)
    acc_ref[...] = acc_ref[...] * scale + contrib
    m_s[0] = m_new
    d_s[0] = d_new

    @pl.when(sb == nsb - 1)
    def _finish():
        d = d_s[0]
        m = m_s[0]
        pooled_ref[...] = jnp.where(
            d > 0, acc_ref[...] / jnp.maximum(d, 1e-30), 0.0)[None]
        m_ref[...] = jnp.where(jnp.isfinite(m), m, 0.0).reshape(1, 1, 1)
        d_ref[...] = d.reshape(1, 1, 1)


def _probs_body(t_spans, seq, spans_ref, sc_ref, m_ref, d_ref, out_ref):
    b = pl.program_id(0)
    sc = sc_ref[...]
    pos = jax.lax.broadcasted_iota(jnp.int32, (1, seq, 1), 1)
    mask = jnp.zeros((1, seq, 1), dtype=jnp.bool_)
    for t in range(t_spans):
        s0 = spans_ref[b * 2 * t_spans + 2 * t]
        e0 = spans_ref[b * 2 * t_spans + 2 * t + 1]
        mask = mask | ((pos >= s0) & (pos < e0))
    m = m_ref[...]
    d = d_ref[...]
    e = jnp.where(mask, jnp.exp(sc - m), 0.0)
    out_ref[...] = jnp.where(d > 0, e / jnp.maximum(d, 1e-30), 0.0)


def _proj_body(p_ref, w_ref, b_ref, o_ref):
    o_ref[...] = jnp.tanh(
        jnp.dot(p_ref[...], w_ref[...], preferred_element_type=jnp.float32)
        + b_ref[...])


def _attention_pool(hidden, spans, pooled_entities, w1, w2, ba):
    b, s, h = hidden.shape
    t_spans = spans.shape[0] // (2 * b)
    nsb = s // _BS
    grid_spec = pltpu.PrefetchScalarGridSpec(
        num_scalar_prefetch=1,
        grid=(b, nsb),
        in_specs=[
            pl.BlockSpec((1, _BS, h), lambda i, j, sp: (i, j, 0)),
            pl.BlockSpec((h, 1), lambda i, j, sp: (0, 0)),
            pl.BlockSpec((h, 1), lambda i, j, sp: (0, 0)),
            pl.BlockSpec((1, 1, h), lambda i, j, sp: (i, 0, 0)),
            pl.BlockSpec((1, 1), lambda i, j, sp: (0, 0)),
        ],
        out_specs=[
            pl.BlockSpec((1, _BS, 1), lambda i, j, sp: (i, j, 0)),
            pl.BlockSpec((1, 1, 1), lambda i, j, sp: (i, 0, 0)),
            pl.BlockSpec((1, 1, 1), lambda i, j, sp: (i, 0, 0)),
            pl.BlockSpec((1, 1, h), lambda i, j, sp: (i, 0, 0)),
        ],
        scratch_shapes=[
            pltpu.VMEM((1, h), jnp.float32),
            pltpu.SMEM((1,), jnp.float32),
            pltpu.SMEM((1,), jnp.float32),
        ],
    )
    return pl.pallas_call(
        functools.partial(_pool_body, nsb, t_spans),
        grid_spec=grid_spec,
        out_shape=[
            jax.ShapeDtypeStruct((b, s, 1), jnp.float32),
            jax.ShapeDtypeStruct((b, 1, 1), jnp.float32),
            jax.ShapeDtypeStruct((b, 1, 1), jnp.float32),
            jax.ShapeDtypeStruct((b, 1, h), jnp.float32),
        ],
        compiler_params=pltpu.CompilerParams(
            dimension_semantics=("arbitrary", "arbitrary")),
    )(spans, hidden, w1, w2, pooled_entities[:, None, :], ba)


def _probs(scores, m, d, spans):
    b, s, _ = scores.shape
    t_spans = spans.shape[0] // (2 * b)
    grid_spec = pltpu.PrefetchScalarGridSpec(
        num_scalar_prefetch=1,
        grid=(b,),
        in_specs=[
            pl.BlockSpec((1, s, 1), lambda i, sp: (i, 0, 0)),
            pl.BlockSpec((1, 1, 1), lambda i, sp: (i, 0, 0)),
            pl.BlockSpec((1, 1, 1), lambda i, sp: (i, 0, 0)),
        ],
        out_specs=pl.BlockSpec((1, s, 1), lambda i, sp: (i, 0, 0)),
    )
    return pl.pallas_call(
        functools.partial(_probs_body, t_spans, s),
        grid_spec=grid_spec,
        out_shape=jax.ShapeDtypeStruct((b, s, 1), jnp.float32),
    )(spans, scores, m, d)


def _project(pooled, w_out, b_out):
    b, h = pooled.shape
    out = w_out.shape[1]
    return pl.pallas_call(
        _proj_body,
        out_shape=jax.ShapeDtypeStruct((b, out), jnp.float32),
    )(pooled, w_out, b_out)


def kernel(hidden, token_idxs, pooled_entities, W_align, b_align, W_out, b_out):
    b, s, h = hidden.shape
    f_ent = token_idxs.shape[0]
    w1 = W_align[:h]
    w2 = W_align[h:]
    ba = b_align.reshape(1, 1).astype(jnp.float32)
    pooled_list = []
    attn_list = []
    for f in range(f_ent):
        spans = token_idxs[f].astype(jnp.int32).reshape(-1)
        scores, m, d, pooled = _attention_pool(
            hidden, spans, pooled_entities, w1, w2, ba)
        probs = _probs(scores, m, d, spans)
        pooled_list.append(pooled[:, 0, :])
        attn_list.append(probs)
    all_pooled = jnp.concatenate(pooled_list, axis=1)
    projected = _project(all_pooled, W_out, b_out.reshape(1, -1))
    return projected, jnp.stack(attn_list, axis=0)


# speedup vs baseline: 1.4131x; 1.0801x over previous
import functools

import jax
import jax.numpy as jnp
from jax.experimental import pallas as pl
from jax.experimental.pallas import tpu as pltpu

_BS = 512


def _fused_body(nsb, t_spans, seq, with_proj, spans_ref,
                hid_ref, w1_ref, w2_ref, pe_ref, ba_ref, wout_ref, bout_ref,
                probs_ref, pooled_ref, proj_ref,
                scores_s, acc_ref, m_s, d_s):
    b = pl.program_id(0)
    sb = pl.program_id(1)

    @pl.when(sb == 0)
    def _init():
        acc_ref[...] = jnp.zeros_like(acc_ref)
        m_s[0] = -jnp.inf
        d_s[0] = 0.0

    @pl.when(sb < nsb)
    def _accumulate():
        hid = hid_ref[0]
        sc = jnp.dot(hid, w2_ref[...], preferred_element_type=jnp.float32)
        pedot = jnp.dot(pe_ref[0], w1_ref[...],
                        preferred_element_type=jnp.float32)
        sc = sc + pedot + ba_ref[0, 0]
        scores_s[pl.ds(sb * _BS, _BS), :] = sc

        pos = sb * _BS + jax.lax.broadcasted_iota(jnp.int32, (_BS, 1), 0)
        mask = jnp.zeros((_BS, 1), dtype=jnp.bool_)
        for t in range(t_spans):
            s0 = spans_ref[b * 2 * t_spans + 2 * t]
            e0 = spans_ref[b * 2 * t_spans + 2 * t + 1]
            mask = mask | ((pos >= s0) & (pos < e0))

        sc_masked = jnp.where(mask, sc, -jnp.inf)
        m_old = m_s[0]
        m_new = jnp.maximum(m_old, jnp.max(sc_masked))
        m_safe = jnp.where(jnp.isfinite(m_new), m_new, 0.0)
        scale = jnp.exp(m_old - m_safe)
        e = jnp.where(mask, jnp.exp(sc - m_safe), 0.0)
        d_s[0] = d_s[0] * scale + jnp.sum(e)
        contrib = jax.lax.dot_general(e, hid, (((0,), (0,)), ((), ())),
                                      preferred_element_type=jnp.float32)
        acc_ref[...] = acc_ref[...] * scale + contrib
        m_s[0] = m_new

    @pl.when(sb == nsb)
    def _finalize():
        m = m_s[0]
        d = d_s[0]
        m_safe = jnp.where(jnp.isfinite(m), m, 0.0)
        sc_full = scores_s[...]
        pos = jax.lax.broadcasted_iota(jnp.int32, (seq, 1), 0)
        mask = jnp.zeros((seq, 1), dtype=jnp.bool_)
        for t in range(t_spans):
            s0 = spans_ref[b * 2 * t_spans + 2 * t]
            e0 = spans_ref[b * 2 * t_spans + 2 * t + 1]
            mask = mask | ((pos >= s0) & (pos < e0))
        e = jnp.where(mask, jnp.exp(sc_full - m_safe), 0.0)
        probs_ref[...] = jnp.where(d > 0, e / jnp.maximum(d, 1e-30), 0.0)[None]
        pooled = jnp.where(d > 0, acc_ref[...] / jnp.maximum(d, 1e-30), 0.0)
        pooled_ref[...] = pooled[None]
        if with_proj:
            proj = jnp.tanh(
                jnp.dot(pooled, wout_ref[...],
                        preferred_element_type=jnp.float32) + bout_ref[...])
            proj_ref[...] = proj[None]


def _proj_body(p_ref, w_ref, b_ref, o_ref):
    o_ref[...] = jnp.tanh(
        jnp.dot(p_ref[...], w_ref[...], preferred_element_type=jnp.float32)
        + b_ref[...])


def _attention_pool(hidden, spans, pooled_entities, w1, w2, ba, wout, bout,
                    with_proj):
    b, s, h = hidden.shape
    out = wout.shape[1]
    t_spans = spans.shape[0] // (2 * b)
    nsb = s // _BS
    grid_spec = pltpu.PrefetchScalarGridSpec(
        num_scalar_prefetch=1,
        grid=(b, nsb + 1),
        in_specs=[
            pl.BlockSpec((1, _BS, h),
                         lambda i, j, sp: (i, jnp.minimum(j, nsb - 1), 0)),
            pl.BlockSpec((h, 1), lambda i, j, sp: (0, 0)),
            pl.BlockSpec((h, 1), lambda i, j, sp: (0, 0)),
            pl.BlockSpec((1, 1, h), lambda i, j, sp: (i, 0, 0)),
            pl.BlockSpec((1, 1), lambda i, j, sp: (0, 0)),
            pl.BlockSpec((h, out), lambda i, j, sp: (0, 0)),
            pl.BlockSpec((1, out), lambda i, j, sp: (0, 0)),
        ],
        out_specs=[
            pl.BlockSpec((1, s, 1), lambda i, j, sp: (i, 0, 0)),
            pl.BlockSpec((1, 1, h), lambda i, j, sp: (i, 0, 0)),
            pl.BlockSpec((1, 1, out), lambda i, j, sp: (i, 0, 0)),
        ],
        scratch_shapes=[
            pltpu.VMEM((s, 1), jnp.float32),
            pltpu.VMEM((1, h), jnp.float32),
            pltpu.SMEM((1,), jnp.float32),
            pltpu.SMEM((1,), jnp.float32),
        ],
    )
    return pl.pallas_call(
        functools.partial(_fused_body, nsb, t_spans, s, with_proj),
        grid_spec=grid_spec,
        out_shape=[
            jax.ShapeDtypeStruct((b, s, 1), jnp.float32),
            jax.ShapeDtypeStruct((b, 1, h), jnp.float32),
            jax.ShapeDtypeStruct((b, 1, out), jnp.float32),
        ],
        compiler_params=pltpu.CompilerParams(
            dimension_semantics=("arbitrary", "arbitrary")),
    )(spans, hidden, w1, w2, pooled_entities[:, None, :], ba, wout, bout)


def _project(pooled, w_out, b_out):
    b, _ = pooled.shape
    out = w_out.shape[1]
    return pl.pallas_call(
        _proj_body,
        out_shape=jax.ShapeDtypeStruct((b, out), jnp.float32),
    )(pooled, w_out, b_out)


def kernel(hidden, token_idxs, pooled_entities, W_align, b_align, W_out, b_out):
    b, s, h = hidden.shape
    f_ent = token_idxs.shape[0]
    w1 = W_align[:h]
    w2 = W_align[h:]
    ba = b_align.reshape(1, 1).astype(jnp.float32)
    bout = b_out.reshape(1, -1)
    pooled_list = []
    attn_list = []
    proj = None
    for f in range(f_ent):
        spans = token_idxs[f].astype(jnp.int32).reshape(-1)
        wout_f = W_out[f * h:(f + 1) * h]
        probs, pooled, proj_f = _attention_pool(
            hidden, spans, pooled_entities, w1, w2, ba, wout_f, bout,
            with_proj=(f_ent == 1))
        pooled_list.append(pooled[:, 0, :])
        attn_list.append(probs)
        proj = proj_f[:, 0, :]
    if f_ent != 1:
        all_pooled = jnp.concatenate(pooled_list, axis=1)
        proj = _project(all_pooled, W_out, bout)
    return proj, jnp.stack(attn_list, axis=0)


# speedup vs baseline: 1.9465x; 1.3775x over previous
import functools

import jax
import jax.numpy as jnp
from jax.experimental import pallas as pl
from jax.experimental.pallas import tpu as pltpu

_BS = 512


def _fused_body(nsb, t_spans, seq, with_proj, spans_ref,
                hid_ref, w1_ref, w2_ref, pe_ref, ba_ref, wout_ref, bout_ref,
                probs_ref, pooled_ref, proj_ref,
                scores_s, acc_ref, m_s, d_s):
    b = pl.program_id(0)
    sb = pl.program_id(1)

    @pl.when(sb == 0)
    def _init():
        acc_ref[...] = jnp.zeros_like(acc_ref)
        m_s[0] = -jnp.inf
        d_s[0] = 0.0

    @pl.when(sb < nsb)
    def _accumulate():
        hid = hid_ref[0]
        sc = jax.lax.dot_general(w2_ref[...], hid, (((1,), (1,)), ((), ())),
                                 preferred_element_type=jnp.float32)
        pedot = jax.lax.dot_general(pe_ref[0], w1_ref[...],
                                    (((1,), (1,)), ((), ())),
                                    preferred_element_type=jnp.float32)
        sc = sc + pedot + ba_ref[0, 0]
        scores_s[:, pl.ds(sb * _BS, _BS)] = sc

        pos = sb * _BS + jax.lax.broadcasted_iota(jnp.int32, (1, _BS), 1)
        mask = jnp.zeros((1, _BS), dtype=jnp.bool_)
        for t in range(t_spans):
            s0 = spans_ref[b * 2 * t_spans + 2 * t]
            e0 = spans_ref[b * 2 * t_spans + 2 * t + 1]
            mask = mask | ((pos >= s0) & (pos < e0))

        sc_masked = jnp.where(mask, sc, -jnp.inf)
        m_old = m_s[0]
        m_new = jnp.maximum(m_old, jnp.max(sc_masked))
        m_safe = jnp.where(jnp.isfinite(m_new), m_new, 0.0)
        scale = jnp.exp(m_old - m_safe)
        e = jnp.where(mask, jnp.exp(sc - m_safe), 0.0)
        d_s[0] = d_s[0] * scale + jnp.sum(e)
        contrib = jnp.dot(e, hid, preferred_element_type=jnp.float32)
        acc_ref[...] = acc_ref[...] * scale + contrib
        m_s[0] = m_new

    @pl.when(sb == nsb)
    def _finalize():
        m = m_s[0]
        d = d_s[0]
        m_safe = jnp.where(jnp.isfinite(m), m, 0.0)
        sc_full = scores_s[...]
        pos = jax.lax.broadcasted_iota(jnp.int32, (1, seq), 1)
        mask = jnp.zeros((1, seq), dtype=jnp.bool_)
        for t in range(t_spans):
            s0 = spans_ref[b * 2 * t_spans + 2 * t]
            e0 = spans_ref[b * 2 * t_spans + 2 * t + 1]
            mask = mask | ((pos >= s0) & (pos < e0))
        e = jnp.where(mask, jnp.exp(sc_full - m_safe), 0.0)
        probs_ref[...] = jnp.where(d > 0, e / jnp.maximum(d, 1e-30), 0.0)[None]
        pooled = jnp.where(d > 0, acc_ref[...] / jnp.maximum(d, 1e-30), 0.0)
        pooled_ref[...] = pooled[None]
        if with_proj:
            proj = jnp.tanh(
                jnp.dot(pooled, wout_ref[...],
                        preferred_element_type=jnp.float32) + bout_ref[...])
            proj_ref[...] = proj[None]


def _proj_body(p_ref, w_ref, b_ref, o_ref):
    o_ref[...] = jnp.tanh(
        jnp.dot(p_ref[...], w_ref[...], preferred_element_type=jnp.float32)
        + b_ref[...])


def _attention_pool(hidden, spans, pooled_entities, w1, w2, ba, wout, bout,
                    with_proj):
    b, s, h = hidden.shape
    out = wout.shape[1]
    t_spans = spans.shape[0] // (2 * b)
    nsb = s // _BS
    grid_spec = pltpu.PrefetchScalarGridSpec(
        num_scalar_prefetch=1,
        grid=(b, nsb + 1),
        in_specs=[
            pl.BlockSpec((1, _BS, h),
                         lambda i, j, sp: (i, jnp.minimum(j, nsb - 1), 0)),
            pl.BlockSpec((1, h), lambda i, j, sp: (0, 0)),
            pl.BlockSpec((1, h), lambda i, j, sp: (0, 0)),
            pl.BlockSpec((1, 1, h), lambda i, j, sp: (i, 0, 0)),
            pl.BlockSpec((1, 1), lambda i, j, sp: (0, 0)),
            pl.BlockSpec((h, out), lambda i, j, sp: (0, 0)),
            pl.BlockSpec((1, out), lambda i, j, sp: (0, 0)),
        ],
        out_specs=[
            pl.BlockSpec((1, 1, s), lambda i, j, sp: (i, 0, 0)),
            pl.BlockSpec((1, 1, h), lambda i, j, sp: (i, 0, 0)),
            pl.BlockSpec((1, 1, out), lambda i, j, sp: (i, 0, 0)),
        ],
        scratch_shapes=[
            pltpu.VMEM((1, s), jnp.float32),
            pltpu.VMEM((1, h), jnp.float32),
            pltpu.SMEM((1,), jnp.float32),
            pltpu.SMEM((1,), jnp.float32),
        ],
    )
    return pl.pallas_call(
        functools.partial(_fused_body, nsb, t_spans, s, with_proj),
        grid_spec=grid_spec,
        out_shape=[
            jax.ShapeDtypeStruct((b, 1, s), jnp.float32),
            jax.ShapeDtypeStruct((b, 1, h), jnp.float32),
            jax.ShapeDtypeStruct((b, 1, out), jnp.float32),
        ],
        compiler_params=pltpu.CompilerParams(
            dimension_semantics=("arbitrary", "arbitrary")),
    )(spans, hidden, w1, w2, pooled_entities[:, None, :], ba, wout, bout)


def _project(pooled, w_out, b_out):
    b, _ = pooled.shape
    out = w_out.shape[1]
    return pl.pallas_call(
        _proj_body,
        out_shape=jax.ShapeDtypeStruct((b, out), jnp.float32),
    )(pooled, w_out, b_out)


def kernel(hidden, token_idxs, pooled_entities, W_align, b_align, W_out, b_out):
    b, s, h = hidden.shape
    f_ent = token_idxs.shape[0]
    w1 = W_align[:h].reshape(1, h)
    w2 = W_align[h:].reshape(1, h)
    ba = b_align.reshape(1, 1).astype(jnp.float32)
    bout = b_out.reshape(1, -1)
    pooled_list = []
    attn_list = []
    proj = None
    for f in range(f_ent):
        spans = token_idxs[f].astype(jnp.int32).reshape(-1)
        wout_f = W_out[f * h:(f + 1) * h]
        probs, pooled, proj_f = _attention_pool(
            hidden, spans, pooled_entities, w1, w2, ba, wout_f, bout,
            with_proj=(f_ent == 1))
        pooled_list.append(pooled[:, 0, :])
        attn_list.append(probs.reshape(b, s, 1))
        proj = proj_f[:, 0, :]
    if f_ent != 1:
        all_pooled = jnp.concatenate(pooled_list, axis=1)
        proj = _project(all_pooled, W_out, bout)
    return proj, jnp.stack(attn_list, axis=0)


# speedup vs baseline: 2.6461x; 1.3594x over previous
import functools

import jax
import jax.numpy as jnp
from jax.experimental import pallas as pl
from jax.experimental.pallas import tpu as pltpu


def _fused_body(t_spans, seq, with_proj, spans_ref,
                hid_ref, w1_ref, w2_ref, pe_ref, ba_ref, wout_ref, bout_ref,
                probs_ref, pooled_ref, proj_ref):
    b = pl.program_id(0)
    hid = hid_ref[0]
    sc = jax.lax.dot_general(w2_ref[...], hid, (((1,), (1,)), ((), ())),
                             preferred_element_type=jnp.float32)
    pedot = jax.lax.dot_general(pe_ref[0], w1_ref[...],
                                (((1,), (1,)), ((), ())),
                                preferred_element_type=jnp.float32)
    sc = sc + pedot + ba_ref[0, 0]

    pos = jax.lax.broadcasted_iota(jnp.int32, (1, seq), 1)
    mask = jnp.zeros((1, seq), dtype=jnp.bool_)
    for t in range(t_spans):
        s0 = spans_ref[b * 2 * t_spans + 2 * t]
        e0 = spans_ref[b * 2 * t_spans + 2 * t + 1]
        mask = mask | ((pos >= s0) & (pos < e0))

    m = jnp.max(jnp.where(mask, sc, -jnp.inf))
    m_safe = jnp.where(jnp.isfinite(m), m, 0.0)
    e = jnp.where(mask, jnp.exp(sc - m_safe), 0.0)
    d = jnp.sum(e)
    probs = jnp.where(d > 0, e / jnp.maximum(d, 1e-30), 0.0)
    probs_ref[...] = probs[None]
    pooled = jnp.dot(probs, hid, preferred_element_type=jnp.float32)
    pooled_ref[...] = pooled[None]
    if with_proj:
        proj = jnp.tanh(
            jnp.dot(pooled, wout_ref[...],
                    preferred_element_type=jnp.float32) + bout_ref[...])
        proj_ref[...] = proj[None]


def _proj_body(p_ref, w_ref, b_ref, o_ref):
    o_ref[...] = jnp.tanh(
        jnp.dot(p_ref[...], w_ref[...], preferred_element_type=jnp.float32)
        + b_ref[...])


def _attention_pool(hidden, spans, pooled_entities, w1, w2, ba, wout, bout,
                    with_proj):
    b, s, h = hidden.shape
    out = wout.shape[1]
    t_spans = spans.shape[0] // (2 * b)
    grid_spec = pltpu.PrefetchScalarGridSpec(
        num_scalar_prefetch=1,
        grid=(b,),
        in_specs=[
            pl.BlockSpec((1, s, h), lambda i, sp: (i, 0, 0)),
            pl.BlockSpec((1, h), lambda i, sp: (0, 0)),
            pl.BlockSpec((1, h), lambda i, sp: (0, 0)),
            pl.BlockSpec((1, 1, h), lambda i, sp: (i, 0, 0)),
            pl.BlockSpec((1, 1), lambda i, sp: (0, 0)),
            pl.BlockSpec((h, out), lambda i, sp: (0, 0)),
            pl.BlockSpec((1, out), lambda i, sp: (0, 0)),
        ],
        out_specs=[
            pl.BlockSpec((1, 1, s), lambda i, sp: (i, 0, 0)),
            pl.BlockSpec((1, 1, h), lambda i, sp: (i, 0, 0)),
            pl.BlockSpec((1, 1, out), lambda i, sp: (i, 0, 0)),
        ],
    )
    return pl.pallas_call(
        functools.partial(_fused_body, t_spans, s, with_proj),
        grid_spec=grid_spec,
        out_shape=[
            jax.ShapeDtypeStruct((b, 1, s), jnp.float32),
            jax.ShapeDtypeStruct((b, 1, h), jnp.float32),
            jax.ShapeDtypeStruct((b, 1, out), jnp.float32),
        ],
        compiler_params=pltpu.CompilerParams(
            dimension_semantics=("arbitrary",)),
    )(spans, hidden, w1, w2, pooled_entities[:, None, :], ba, wout, bout)


def _project(pooled, w_out, b_out):
    b, _ = pooled.shape
    out = w_out.shape[1]
    return pl.pallas_call(
        _proj_body,
        out_shape=jax.ShapeDtypeStruct((b, out), jnp.float32),
    )(pooled, w_out, b_out)


def kernel(hidden, token_idxs, pooled_entities, W_align, b_align, W_out, b_out):
    b, s, h = hidden.shape
    f_ent = token_idxs.shape[0]
    w1 = W_align[:h].reshape(1, h)
    w2 = W_align[h:].reshape(1, h)
    ba = b_align.reshape(1, 1).astype(jnp.float32)
    bout = b_out.reshape(1, -1)
    pooled_list = []
    attn_list = []
    proj = None
    for f in range(f_ent):
        spans = token_idxs[f].astype(jnp.int32).reshape(-1)
        wout_f = W_out[f * h:(f + 1) * h]
        probs, pooled, proj_f = _attention_pool(
            hidden, spans, pooled_entities, w1, w2, ba, wout_f, bout,
            with_proj=(f_ent == 1))
        pooled_list.append(pooled[:, 0, :])
        attn_list.append(probs.reshape(b, s, 1))
        proj = proj_f[:, 0, :]
    if f_ent != 1:
        all_pooled = jnp.concatenate(pooled_list, axis=1)
        proj = _project(all_pooled, W_out, bout)
    return proj, jnp.stack(attn_list, axis=0)
